# Initial kernel scaffold; baseline (speedup 1.0000x reference)
import jax
import jax.numpy as jnp
from jax.experimental import pallas as pl


def kernel(q, k, v, edge_index):
    raise NotImplementedError("write your pallas kernel here")


# speedup vs baseline: 6.3018x; 6.3018x over previous
import jax
import jax.numpy as jnp
from jax import lax
from jax.experimental import pallas as pl
from jax.experimental.pallas import tpu as pltpu
from jax.experimental.pallas import tpu_sc as plsc

N = 10000
E = 160000
H = 8
DK = 32
ROW = H * DK
PARTS = 3

NC = 2
NS = 16

HALF = N // NC
ACC_ROWS = 15104
ZERO_CHUNK = ACC_ROWS // NS
E_PER_TILE = E // NS
C = 8
NBLK = E_PER_TILE // C
INV_SQRT_DK = 1.0 / float(DK) ** 0.5


def _body(qf, kf, vf, src, dst, zsrc, out,
          srcb, dstb, dstb16, idxb, kb, qb, vb, vs, acc, sem):
  c = lax.axis_index("c")
  s = lax.axis_index("s")

  pltpu.sync_copy(zsrc.at[pl.ds(s * ZERO_CHUNK, ZERO_CHUNK)],
                  acc.at[pl.ds(s * ZERO_CHUNK, ZERO_CHUNK)])

  zero16 = jnp.zeros((16,), jnp.float32)
  for e in range(C):
    for i in range(1, 8):
      vs[2, e, pl.ds(16 * i, 16)] = zero16
  plsc.subcore_barrier()

  dst_base = c * HALF
  lanes = lax.iota(jnp.int32, 16)

  def edge_fn(e, _):
    zv = jnp.zeros((16,), jnp.float32)
    for h in range(H):
      a0 = kb[e, pl.ds(32 * h, 16)] * qb[e, pl.ds(32 * h, 16)]
      a1 = kb[e, pl.ds(32 * h + 16, 16)] * qb[e, pl.ds(32 * h + 16, 16)]
      tot = jnp.sum(a0 + a1)
      sc = jnp.minimum(jnp.maximum(tot * INV_SQRT_DK, -5.0), 5.0)
      ev = jnp.exp(jnp.broadcast_to(sc, (16,)))
      zv = jnp.where(lanes == h, ev, zv)
      for t in range(2):
        col = 32 * h + 16 * t
        vs[col // 128, e, pl.ds(col % 128, 16)] = vb[e, pl.ds(col, 16)] * ev
    vs[2, e, pl.ds(0, 16)] = jnp.where(lanes < H, zv, 0.0)
    return 0

  def blk_fn(b, _):
    e0 = s * E_PER_TILE + b * C
    pltpu.sync_copy(src.at[pl.ds(e0, C)], srcb)
    pltpu.sync_copy(dst.at[pl.ds(e0, C)], dstb)
    pltpu.sync_copy(dst.at[pl.ds(e0, 16)], dstb16)
    d1 = pltpu.async_copy(kf.at[srcb], kb, sem)
    d2 = pltpu.async_copy(qf.at[dstb], qb, sem)
    d3 = pltpu.async_copy(vf.at[srcb], vb, sem)
    d1.wait()
    d2.wait()
    d3.wait()

    dv = dstb16[pl.ds(0, 16)]
    loc = dv - dst_base
    ok = (loc >= 0) & (loc < HALF) & (lanes < C)
    base3 = 3 * jnp.where(ok, loc, HALF)
    inlane = lanes < C
    for j in range(PARTS):
      plsc.store_scatter(idxb, [jnp.full((16,), j, jnp.int32), lanes],
                         base3 + j, mask=inlane)

    lax.fori_loop(0, C, edge_fn, 0)

    for j in range(PARTS):
      pltpu.sync_copy(vs.at[j], acc.at[idxb.at[j]], add=True)
    return 0

  lax.fori_loop(0, NBLK, blk_fn, 0)
  plsc.subcore_barrier()

  r0 = s * ZERO_CHUNK
  pltpu.sync_copy(acc.at[pl.ds(r0, ZERO_CHUNK)],
                  out.at[pl.ds(c * ACC_ROWS + r0, ZERO_CHUNK)])


@jax.jit
def _run(qf, kf, vf, src, dst, zsrc):
  mesh = plsc.VectorSubcoreMesh(core_axis_name="c", subcore_axis_name="s",
                                num_cores=NC, num_subcores=NS)
  return pl.kernel(
      _body,
      out_type=jax.ShapeDtypeStruct((NC * ACC_ROWS, 128), jnp.float32),
      mesh=mesh,
      compiler_params=pltpu.CompilerParams(needs_layout_passes=False),
      scratch_types=[
          pltpu.VMEM((C,), jnp.int32),
          pltpu.VMEM((C,), jnp.int32),
          pltpu.VMEM((16,), jnp.int32),
          pltpu.VMEM((PARTS, C), jnp.int32),
          pltpu.VMEM((C, ROW), jnp.float32),
          pltpu.VMEM((C, ROW), jnp.float32),
          pltpu.VMEM((C, ROW), jnp.float32),
          pltpu.VMEM((PARTS, C, 128), jnp.float32),
          pltpu.VMEM_SHARED((ACC_ROWS, 128), jnp.float32),
          pltpu.SemaphoreType.DMA,
      ],
  )(qf, kf, vf, src, dst, zsrc)


def kernel(q, k, v, edge_index):
  qf = q.reshape(N, ROW)
  kf = k.reshape(N, ROW)
  vf = v.reshape(N, ROW)
  pad = jnp.zeros((2, 16), jnp.int32)
  ei = jnp.concatenate([edge_index, pad], axis=1)
  src = ei[0]
  dst = ei[1]
  zsrc = jnp.zeros((ACC_ROWS, 128), jnp.float32)
  out = _run(qf, kf, vf, src, dst, zsrc)
  halves = [
      out[c * ACC_ROWS: c * ACC_ROWS + PARTS * HALF].reshape(HALF, PARTS * 128)
      for c in range(NC)
  ]
  full = jnp.concatenate(halves, axis=0)
  wv = full[:, :ROW].reshape(N, H, DK)
  z = full[:, ROW:ROW + H].reshape(N, H, 1)
  return wv, z
